# speedup vs baseline: 4.4107x; 4.4107x over previous
import functools

import jax
import jax.numpy as jnp
from jax import lax
from jax.experimental import pallas as pl
from jax.experimental.pallas import tpu as pltpu
from jax.experimental.pallas import tpu_sc as plsc

VOCAB1 = 100001
EMBED = 128
HIDDEN = 512
OUT = 128
B = 4096
L = 50

NC = 2
NS = 16
NW = NC * NS
ROWS_PER_W = B // NW
IDX_PER_W = ROWS_PER_W * L
CHUNK = 128
NCHUNK = IDX_PER_W // CHUNK


def _sc_pool_body(x_r, table, out_hbm, idx_v, buf0, buf1, out_v, sem0, sem1):
    wid = lax.axis_index("s") * NC + lax.axis_index("c")

    pltpu.sync_copy(x_r.at[wid], idx_v)

    zero = jnp.zeros((16,), jnp.float32)

    def zero_body(r, _):
        for c in range(EMBED // 16):
            out_v[r, pl.ds(c * 16, 16)] = zero
        return 0

    lax.fori_loop(0, ROWS_PER_W, zero_body, 0)

    pltpu.async_copy(table.at[idx_v.at[0]], buf0, sem0)
    pltpu.async_copy(table.at[idx_v.at[1]], buf1, sem1)

    def accum(buf, carry):
        def body(r, rc):
            row, cnt = rc
            for c in range(EMBED // 16):
                v = buf[r, pl.ds(c * 16, 16)]
                plsc.addupdate(out_v.at[row, pl.ds(c * 16, 16)], v)
            cnt = cnt + 1
            wrap = cnt == L
            row = jnp.where(wrap, row + 1, row)
            cnt = jnp.where(wrap, 0, cnt)
            return row, cnt

        return lax.fori_loop(0, CHUNK, body, carry)

    def pair_body(p, carry):
        pltpu.make_async_copy(table.at[idx_v.at[2 * p]], buf0, sem0).wait()
        carry = accum(buf0, carry)

        @pl.when(p < NCHUNK // 2 - 1)
        def _():
            pltpu.async_copy(table.at[idx_v.at[2 * p + 2]], buf0, sem0)

        pltpu.make_async_copy(table.at[idx_v.at[2 * p + 1]], buf1, sem1).wait()
        carry = accum(buf1, carry)

        @pl.when(p < NCHUNK // 2 - 1)
        def _():
            pltpu.async_copy(table.at[idx_v.at[2 * p + 3]], buf1, sem1)

        return carry

    lax.fori_loop(0, NCHUNK // 2, pair_body, (jnp.int32(0), jnp.int32(0)))

    pltpu.sync_copy(out_v, out_hbm.at[pl.ds(wid * ROWS_PER_W, ROWS_PER_W)])


def _sc_pool(x_r, table):
    mesh = plsc.VectorSubcoreMesh(core_axis_name="c", subcore_axis_name="s")
    return pl.kernel(
        _sc_pool_body,
        out_type=jax.ShapeDtypeStruct((B, EMBED), jnp.float32),
        mesh=mesh,
        scratch_types=[
            pltpu.VMEM((NCHUNK, CHUNK), jnp.int32),
            pltpu.VMEM((CHUNK, EMBED), jnp.float32),
            pltpu.VMEM((CHUNK, EMBED), jnp.float32),
            pltpu.VMEM((ROWS_PER_W, EMBED), jnp.float32),
            pltpu.SemaphoreType.DMA,
            pltpu.SemaphoreType.DMA,
        ],
    )(x_r, table)


BM = 512


def _mlp_body(p_ref, w1_ref, b1_ref, w2_ref, b2_ref, o_ref):
    h = jnp.dot(p_ref[...] * (1.0 / L), w1_ref[...],
                preferred_element_type=jnp.float32)
    h = jnp.maximum(h + b1_ref[...], 0.0)
    o_ref[...] = jnp.dot(h, w2_ref[...],
                         preferred_element_type=jnp.float32) + b2_ref[...]


def _mlp(pooled_sum, W1, b1, W2, b2):
    return pl.pallas_call(
        _mlp_body,
        grid=(B // BM,),
        in_specs=[
            pl.BlockSpec((BM, EMBED), lambda i: (i, 0)),
            pl.BlockSpec((EMBED, HIDDEN), lambda i: (0, 0)),
            pl.BlockSpec((1, HIDDEN), lambda i: (0, 0)),
            pl.BlockSpec((HIDDEN, OUT), lambda i: (0, 0)),
            pl.BlockSpec((1, OUT), lambda i: (0, 0)),
        ],
        out_specs=pl.BlockSpec((BM, OUT), lambda i: (i, 0)),
        out_shape=jax.ShapeDtypeStruct((B, OUT), jnp.float32),
    )(pooled_sum, W1, b1.reshape(1, HIDDEN), W2, b2.reshape(1, OUT))


@jax.jit
def kernel(x, table, W1, b1, W2, b2):
    x_r = x.astype(jnp.int32).reshape(NW, NCHUNK, CHUNK)
    pooled_sum = _sc_pool(x_r, table)
    return _mlp(pooled_sum, W1, b1, W2, b2)
